# Initial kernel scaffold; baseline (speedup 1.0000x reference)
import jax
import jax.numpy as jnp
from jax.experimental import pallas as pl


def kernel(h, x, edge_index, edge_attr, W_e1, b_e1, W_e2, b_e2, W_n1, b_n1, W_n2, b_n2, W_c1, b_c1, W_c2, W_a, b_a):
    raise NotImplementedError("write your pallas kernel here")


# speedup vs baseline: 2.4336x; 2.4336x over previous
import functools

import jax
import jax.numpy as jnp
from jax import lax
from jax.experimental import pallas as pl
from jax.experimental.pallas import tpu as pltpu
from jax.experimental.pallas import tpu_sc as plsc

N = 10000
E = 160000
D = 256
ED = 16
XL = 16
HD = D // 2

NC, NS = 2, 16
NW = NC * NS

GEB = 200
SEB = 80
TMPW = 512
E_PER_W = E // NW
E_PER_TILE = E // NS
NPT = 624
NTAIL = N - NS * NPT

_mesh = plsc.VectorSubcoreMesh(core_axis_name="c", subcore_axis_name="s")
_sc_params = pltpu.CompilerParams(needs_layout_passes=False)


def _pre_body(h_ref, w1a_ref, w1b_ref, a_ref, b_ref):
    h = h_ref[...]
    a_ref[...] = jnp.dot(h, w1a_ref[...], preferred_element_type=jnp.float32)
    b_ref[...] = jnp.dot(h, w1b_ref[...], preferred_element_type=jnp.float32)


def _pre(h, w1a, w1b):
    bn = 2000
    return pl.pallas_call(
        _pre_body,
        grid=(N // bn,),
        in_specs=[
            pl.BlockSpec((bn, D), lambda i: (i, 0)),
            pl.BlockSpec((D, D), lambda i: (0, 0)),
            pl.BlockSpec((D, D), lambda i: (0, 0)),
        ],
        out_specs=[
            pl.BlockSpec((bn, D), lambda i: (i, 0)),
            pl.BlockSpec((bn, D), lambda i: (i, 0)),
        ],
        out_shape=[
            jax.ShapeDtypeStruct((N, D), jnp.float32),
            jax.ShapeDtypeStruct((N, D), jnp.float32),
        ],
    )(h, w1a, w1b)


@functools.partial(
    pl.kernel,
    mesh=_mesh,
    out_type=(
        jax.ShapeDtypeStruct((E, D), jnp.float32),
        jax.ShapeDtypeStruct((E, D), jnp.float32),
        jax.ShapeDtypeStruct((E * XL,), jnp.float32),
    ),
    scratch_types=[
        pltpu.VMEM((GEB,), jnp.int32),
        pltpu.VMEM((GEB,), jnp.int32),
        pltpu.VMEM((GEB, D), jnp.float32),
        pltpu.VMEM((GEB * XL,), jnp.float32),
        pltpu.VMEM((N * 4,), jnp.float32),
    ],
    compiler_params=_sc_params,
)
def _gather_k(a_hbm, b_hbm, x4_hbm, row_hbm, col_hbm,
              arow_hbm, bcol_hbm, xrel_hbm,
              rbuf, cbuf, abuf, xrelbuf, xv):
    wid = lax.axis_index("s") * NC + lax.axis_index("c")
    base = wid * E_PER_W

    pltpu.sync_copy(x4_hbm, xv)

    def zbody(i, carry):
        xrelbuf[pl.ds(i * 16, 16)] = jnp.zeros((16,), jnp.float32)
        return carry
    lax.fori_loop(0, GEB * XL // 16, zbody, 0)

    iota = lax.iota(jnp.int32, 16)
    tail_mask = iota >= 8

    def body(j, carry):
        off = base + j * GEB
        pltpu.sync_copy(row_hbm.at[pl.ds(off, GEB)], rbuf)
        pltpu.sync_copy(col_hbm.at[pl.ds(off, GEB)], cbuf)
        pltpu.sync_copy(a_hbm.at[rbuf], abuf)
        pltpu.sync_copy(abuf, arow_hbm.at[pl.ds(off, GEB)])
        pltpu.sync_copy(b_hbm.at[cbuf], abuf)
        pltpu.sync_copy(abuf, bcol_hbm.at[pl.ds(off, GEB)])
        for g in range(13):
            g0 = g * 16 if g < 12 else GEB - 16
            msk = None if g < 12 else tail_mask
            idr = rbuf[pl.ds(g0, 16)]
            idc = cbuf[pl.ds(g0, 16)]
            if msk is not None:
                idr = jnp.where(msk, idr, 0)
                idc = jnp.where(msk, idc, 0)
            idr4 = idr * 4
            idc4 = idc * 4
            rowv = (g0 + iota) * XL
            d2 = jnp.zeros((16,), jnp.float32)
            for comp in range(3):
                xr = plsc.load_gather(xv, [idr4 + comp], mask=msk)
                xc = plsc.load_gather(xv, [idc4 + comp], mask=msk)
                rel = xr - xc
                d2 = d2 + rel * rel
                plsc.store_scatter(xrelbuf, [rowv + comp], rel, mask=msk)
            plsc.store_scatter(xrelbuf, [rowv + 3], d2, mask=msk)
        pltpu.sync_copy(xrelbuf, xrel_hbm.at[pl.ds(off * XL, GEB * XL)])
        return carry

    lax.fori_loop(0, E_PER_W // GEB, body, 0)


def _edge_body(arow_ref, bcol_ref, xrel_ref, ea_ref,
               wea_ref, be1_ref, we2_ref, be2_ref, wa_ref, ba_ref,
               wc1_ref, bc1_ref, wc2_ref, wd_ref,
               payl_ref, payr_ref, cpay_ref):
    xrel = xrel_ref[...]
    lane = lax.broadcasted_iota(jnp.int32, (1, XL), 1)
    rel = jnp.where(lane < 3, xrel, 0.0)
    d2 = xrel[:, 3:4]
    dist = jnp.sqrt(d2)
    u = arow_ref[...] + bcol_ref[...]
    u += dist * wd_ref[...]
    u += jnp.dot(ea_ref[...], wea_ref[...], preferred_element_type=jnp.float32)
    u += be1_ref[...]
    t = u * jax.nn.sigmoid(u)
    v = jnp.dot(t, we2_ref[...], preferred_element_type=jnp.float32) + be2_ref[...]
    m = v * jax.nn.sigmoid(v)
    att = jax.nn.sigmoid(
        jnp.sum(m * wa_ref[...], axis=1, keepdims=True) + ba_ref[...])
    pay = att * m
    payl_ref[...] = pay[:, :HD]
    payr_ref[...] = pay[:, HD:]
    cv = jnp.dot(m, wc1_ref[...], preferred_element_type=jnp.float32) + bc1_ref[...]
    c1 = cv * jax.nn.sigmoid(cv)
    cw = jnp.sum(c1 * wc2_ref[...], axis=1, keepdims=True)
    cpay_ref[...] = cw * rel / (dist + 1e-8)


def _edge(arow, bcol, xrel, ea, wea, be1, we2, be2, wa, ba, wc1, bc1, wc2, wd):
    be = 2000
    full = lambda shape: pl.BlockSpec(shape, lambda i: (0, 0))
    return pl.pallas_call(
        _edge_body,
        grid=(E // be,),
        in_specs=[
            pl.BlockSpec((be, D), lambda i: (i, 0)),
            pl.BlockSpec((be, D), lambda i: (i, 0)),
            pl.BlockSpec((be, XL), lambda i: (i, 0)),
            pl.BlockSpec((be, ED), lambda i: (i, 0)),
            full((ED, D)), full((1, D)), full((D, D)), full((1, D)),
            full((1, D)), full((1, 1)), full((D, D)), full((1, D)),
            full((1, D)), full((1, D)),
        ],
        out_specs=[
            pl.BlockSpec((be, HD), lambda i: (i, 0)),
            pl.BlockSpec((be, HD), lambda i: (i, 0)),
            pl.BlockSpec((be, XL), lambda i: (i, 0)),
        ],
        out_shape=[
            jax.ShapeDtypeStruct((E, HD), jnp.float32),
            jax.ShapeDtypeStruct((E, HD), jnp.float32),
            jax.ShapeDtypeStruct((E, XL), jnp.float32),
        ],
    )(arow, bcol, xrel, ea, wea, be1, we2, be2, wa, ba, wc1, bc1, wc2, wd)


@functools.partial(
    pl.kernel,
    mesh=_mesh,
    out_type=(
        jax.ShapeDtypeStruct((N, HD), jnp.float32),
        jax.ShapeDtypeStruct((N, HD), jnp.float32),
        jax.ShapeDtypeStruct((NS * N * 3,), jnp.float32),
        jax.ShapeDtypeStruct((N * 3,), jnp.float32),
    ),
    scratch_types=[
        pltpu.VMEM_SHARED((N, HD), jnp.float32),
        pltpu.VMEM((SEB,), jnp.int32),
        pltpu.VMEM((SEB, HD), jnp.float32),
        pltpu.VMEM((SEB * XL,), jnp.float32),
        pltpu.VMEM((N * 3,), jnp.float32),
        pltpu.VMEM((TMPW,), jnp.int32),
        pltpu.VMEM((2000,), jnp.float32),
        pltpu.VMEM((2000,), jnp.float32),
    ],
    compiler_params=_sc_params,
)
def _scatter_k(payl_hbm, payr_hbm, cpay_hbm, row_hbm,
               ml_hbm, mr_hbm, cout_hbm, cfin_hbm,
               acc_sh, ibuf, pbuf, cbuf, cvacc, tmp, tbuf, sbuf):
    c = lax.axis_index("c")
    s = lax.axis_index("s")
    wid = s * NC + c
    nrow0 = s * NPT

    def zpbody(k, carry):
        pbuf[k // 8, pl.ds((k % 8) * 16, 16)] = jnp.zeros((16,), jnp.float32)
        return carry
    lax.fori_loop(0, SEB * 8, zpbody, 0)

    nfull = NPT // SEB
    nrem = NPT - nfull * SEB
    for z in range(nfull):
        pltpu.sync_copy(pbuf, acc_sh.at[pl.ds(nrow0 + z * SEB, SEB)])
    pltpu.sync_copy(pbuf.at[pl.ds(0, nrem)],
                    acc_sh.at[pl.ds(nrow0 + nfull * SEB, nrem)])

    @pl.when(s == 0)
    def _():
        pltpu.sync_copy(pbuf.at[pl.ds(0, NTAIL)],
                        acc_sh.at[pl.ds(NS * NPT, NTAIL)])

    def zbody(i, carry):
        cvacc[pl.ds(i * 16, 16)] = jnp.zeros((16,), jnp.float32)
        return carry
    lax.fori_loop(0, N * 3 // 16, zbody, 0)

    plsc.subcore_barrier()

    ebase = s * E_PER_TILE
    nblk = E_PER_TILE // SEB

    def body(j, carry):
        off = ebase + j * SEB
        pltpu.sync_copy(row_hbm.at[pl.ds(off, SEB)], ibuf)

        @pl.when(c == 0)
        def _():
            pltpu.sync_copy(payl_hbm.at[pl.ds(off, SEB)], pbuf)

        @pl.when(c == 1)
        def _():
            pltpu.sync_copy(payr_hbm.at[pl.ds(off, SEB)], pbuf)

        pltpu.sync_copy(pbuf, acc_sh.at[ibuf], add=True)

        @pl.when(c == 1)
        def _():
            pltpu.sync_copy(cpay_hbm.at[pl.ds(off * XL, SEB * XL)], cbuf)
            iota = lax.iota(jnp.int32, 16)
            for g in range(SEB // 16):
                g0 = g * 16
                idxv = ibuf[pl.ds(g0, 16)]
                ev = (g0 + iota) * XL
                cpx = plsc.load_gather(cbuf, [ev])
                cpy = plsc.load_gather(cbuf, [ev + 1])
                cpz = plsc.load_gather(cbuf, [ev + 2])
                addr = idxv * 3
                slot = jnp.bitwise_and(idxv, TMPW - 1)

                def wbody(rem):
                    plsc.store_scatter(tmp, [slot], iota, mask=rem)
                    win = jnp.logical_and(
                        plsc.load_gather(tmp, [slot], mask=rem) == iota, rem)
                    plsc.addupdate_scatter(cvacc, [addr], cpx, mask=win)
                    plsc.addupdate_scatter(cvacc, [addr + 1], cpy, mask=win)
                    plsc.addupdate_scatter(cvacc, [addr + 2], cpz, mask=win)
                    return jnp.logical_and(rem, jnp.logical_not(win))

                lax.while_loop(jnp.any, wbody, iota >= 0)

        return carry

    lax.fori_loop(0, nblk, body, 0)

    @pl.when(c == 1)
    def _():
        pltpu.sync_copy(cvacc, cout_hbm.at[pl.ds(s * (N * 3), N * 3)])

    plsc.subcore_barrier()

    @pl.when(jnp.logical_and(c == 1, s < 15))
    def _():
        r0 = s * 2000

        def zsbody(i, carry):
            sbuf[pl.ds(i * 16, 16)] = jnp.zeros((16,), jnp.float32)
            return carry
        lax.fori_loop(0, 125, zsbody, 0)

        for t in range(NS):
            pltpu.sync_copy(cout_hbm.at[pl.ds(t * (N * 3) + r0, 2000)], tbuf)

            def abody(i, carry):
                sl = pl.ds(i * 16, 16)
                sbuf[sl] = sbuf[sl] + tbuf[sl]
                return carry
            lax.fori_loop(0, 125, abody, 0)

        pltpu.sync_copy(sbuf, cfin_hbm.at[pl.ds(r0, 2000)])

    def _writeback(dst_hbm):
        for z in range(nfull):
            pltpu.sync_copy(acc_sh.at[pl.ds(nrow0 + z * SEB, SEB)], pbuf)
            pltpu.sync_copy(pbuf, dst_hbm.at[pl.ds(nrow0 + z * SEB, SEB)])
        pltpu.sync_copy(acc_sh.at[pl.ds(nrow0 + nfull * SEB, nrem)],
                        pbuf.at[pl.ds(0, nrem)])
        pltpu.sync_copy(pbuf.at[pl.ds(0, nrem)],
                        dst_hbm.at[pl.ds(nrow0 + nfull * SEB, nrem)])

        @pl.when(s == 0)
        def _():
            pltpu.sync_copy(acc_sh.at[pl.ds(NS * NPT, NTAIL)],
                            pbuf.at[pl.ds(0, NTAIL)])
            pltpu.sync_copy(pbuf.at[pl.ds(0, NTAIL)],
                            dst_hbm.at[pl.ds(NS * NPT, NTAIL)])

    @pl.when(c == 0)
    def _():
        _writeback(ml_hbm)

    @pl.when(c == 1)
    def _():
        _writeback(mr_hbm)


def _node_body(h_ref, ml_ref, mr_ref, x_ref, cfin_ref,
               wn1a_ref, wn1bl_ref, wn1br_ref, bn1_ref, wn2_ref, bn2_ref,
               hnew_ref, xnew_ref):
    g = (jnp.dot(h_ref[...], wn1a_ref[...], preferred_element_type=jnp.float32)
         + jnp.dot(ml_ref[...], wn1bl_ref[...], preferred_element_type=jnp.float32)
         + jnp.dot(mr_ref[...], wn1br_ref[...], preferred_element_type=jnp.float32)
         + bn1_ref[...])
    g = g * jax.nn.sigmoid(g)
    hnew_ref[...] = (h_ref[...]
                     + jnp.dot(g, wn2_ref[...], preferred_element_type=jnp.float32)
                     + bn2_ref[...])
    xnew_ref[...] = x_ref[...] + cfin_ref[...]


def _node(h, ml, mr, x, cfin, wn1a, wn1bl, wn1br, bn1, wn2, bn2):
    bn = 2000
    full = lambda shape: pl.BlockSpec(shape, lambda i: (0, 0))
    return pl.pallas_call(
        _node_body,
        grid=(N // bn,),
        in_specs=[
            pl.BlockSpec((bn, D), lambda i: (i, 0)),
            pl.BlockSpec((bn, HD), lambda i: (i, 0)),
            pl.BlockSpec((bn, HD), lambda i: (i, 0)),
            pl.BlockSpec((bn, 3), lambda i: (i, 0)),
            pl.BlockSpec((bn, 3), lambda i: (i, 0)),
            full((D, D)), full((HD, D)), full((HD, D)), full((1, D)),
            full((D, D)), full((1, D)),
        ],
        out_specs=[
            pl.BlockSpec((bn, D), lambda i: (i, 0)),
            pl.BlockSpec((bn, 3), lambda i: (i, 0)),
        ],
        out_shape=[
            jax.ShapeDtypeStruct((N, D), jnp.float32),
            jax.ShapeDtypeStruct((N, 3), jnp.float32),
        ],
    )(h, ml, mr, x, cfin, wn1a, wn1bl, wn1br, bn1, wn2, bn2)


def kernel(h, x, edge_index, edge_attr, W_e1, b_e1, W_e2, b_e2,
           W_n1, b_n1, W_n2, b_n2, W_c1, b_c1, W_c2, W_a, b_a):
    row = edge_index[0]
    col = edge_index[1]

    w1a = W_e1[:D]
    w1b = W_e1[D:2 * D]
    wd = W_e1[2 * D:2 * D + 1]
    wea = W_e1[2 * D + 1:]
    be1 = b_e1.reshape(1, D)
    be2 = b_e2.reshape(1, D)
    wa = W_a.reshape(1, D)
    ba = b_a.reshape(1, 1)
    bc1 = b_c1.reshape(1, D)
    wc2 = W_c2.reshape(1, D)
    wn1a = W_n1[:D]
    wn1bl = W_n1[D:D + HD]
    wn1br = W_n1[D + HD:]
    bn1 = b_n1.reshape(1, D)
    bn2 = b_n2.reshape(1, D)

    x4 = jnp.pad(x, ((0, 0), (0, 1))).reshape(N * 4)

    a, b = _pre(h, w1a, w1b)
    arow, bcol, xrel = _gather_k(a, b, x4, row, col)
    payl, payr, cpay = _edge(arow, bcol, xrel.reshape(E, XL), edge_attr,
                             wea, be1, W_e2, be2, wa, ba,
                             W_c1, bc1, wc2, wd)
    ml, mr, _cpart, cfin = _scatter_k(payl, payr, cpay.reshape(E * XL), row)
    hnew, xnew = _node(h, ml, mr, x, cfin.reshape(N, 3),
                       wn1a, wn1bl, wn1br, bn1, W_n2, bn2)
    return hnew, xnew


# speedup vs baseline: 2.9201x; 1.1999x over previous
import functools

import jax
import jax.numpy as jnp
from jax import lax
from jax.experimental import pallas as pl
from jax.experimental.pallas import tpu as pltpu
from jax.experimental.pallas import tpu_sc as plsc

N = 10000
E = 160000
D = 256
ED = 16
XL = 16
HD = D // 2

NC, NS = 2, 16
NW = NC * NS

GEB = 200
SEB = 80
TMPW = 512
E_PER_W = E // NW
E_PER_TILE = E // NS
NPT = 624
NTAIL = N - NS * NPT

_mesh = plsc.VectorSubcoreMesh(core_axis_name="c", subcore_axis_name="s")
_sc_params = pltpu.CompilerParams(needs_layout_passes=False)


def _pack_bf16_pair(m):
    mu = lax.bitcast_convert_type(
        m.astype(jnp.bfloat16), jnp.uint16).astype(jnp.uint32)
    packed = mu[:, :HD] | (mu[:, HD:] << 16)
    return lax.bitcast_convert_type(packed, jnp.int32)


def _unpack_bf16_pair(p):
    lo = lax.bitcast_convert_type(lax.shift_left(p, 16), jnp.float32)
    hi = lax.bitcast_convert_type(
        jnp.bitwise_and(p, jnp.int32(-65536)), jnp.float32)
    return jnp.concatenate([lo, hi], axis=1)


def _pre_body(h_ref, w1a_ref, w1b_ref, a_ref, b_ref):
    h = h_ref[...]
    a_ref[...] = _pack_bf16_pair(
        jnp.dot(h, w1a_ref[...], preferred_element_type=jnp.float32))
    b_ref[...] = _pack_bf16_pair(
        jnp.dot(h, w1b_ref[...], preferred_element_type=jnp.float32))


def _pre(h, w1a, w1b):
    bn = 2000
    return pl.pallas_call(
        _pre_body,
        grid=(N // bn,),
        in_specs=[
            pl.BlockSpec((bn, D), lambda i: (i, 0)),
            pl.BlockSpec((D, D), lambda i: (0, 0)),
            pl.BlockSpec((D, D), lambda i: (0, 0)),
        ],
        out_specs=[
            pl.BlockSpec((bn, HD), lambda i: (i, 0)),
            pl.BlockSpec((bn, HD), lambda i: (i, 0)),
        ],
        out_shape=[
            jax.ShapeDtypeStruct((N, HD), jnp.int32),
            jax.ShapeDtypeStruct((N, HD), jnp.int32),
        ],
    )(h, w1a, w1b)


@functools.partial(
    pl.kernel,
    mesh=_mesh,
    out_type=(
        jax.ShapeDtypeStruct((E, HD), jnp.int32),
        jax.ShapeDtypeStruct((E, HD), jnp.int32),
        jax.ShapeDtypeStruct((E * XL,), jnp.float32),
    ),
    scratch_types=[
        pltpu.VMEM((GEB,), jnp.int32),
        pltpu.VMEM((GEB,), jnp.int32),
        pltpu.VMEM((GEB, HD), jnp.int32),
        pltpu.VMEM((GEB, HD), jnp.int32),
        pltpu.VMEM((GEB * XL,), jnp.float32),
        pltpu.VMEM((N * 4,), jnp.float32),
        pltpu.SemaphoreType.DMA,
    ],
    compiler_params=_sc_params,
)
def _gather_k(a_hbm, b_hbm, x4_hbm, row_hbm, col_hbm,
              arow_hbm, bcol_hbm, xrel_hbm,
              rbuf, cbuf, abuf, bbuf, xrelbuf, xv, sem):
    wid = lax.axis_index("s") * NC + lax.axis_index("c")
    base = wid * E_PER_W

    pltpu.sync_copy(x4_hbm, xv)

    def zbody(i, carry):
        xrelbuf[pl.ds(i * 16, 16)] = jnp.zeros((16,), jnp.float32)
        return carry
    lax.fori_loop(0, GEB * XL // 16, zbody, 0)

    iota = lax.iota(jnp.int32, 16)
    tail_mask = iota >= 8

    def body(j, carry):
        off = base + j * GEB
        pltpu.sync_copy(row_hbm.at[pl.ds(off, GEB)], rbuf)
        pltpu.sync_copy(col_hbm.at[pl.ds(off, GEB)], cbuf)
        ca = pltpu.async_copy(a_hbm.at[rbuf], abuf, sem)
        cb = pltpu.async_copy(b_hbm.at[cbuf], bbuf, sem)
        for g in range(13):
            g0 = g * 16 if g < 12 else GEB - 16
            msk = None if g < 12 else tail_mask
            idr = rbuf[pl.ds(g0, 16)]
            idc = cbuf[pl.ds(g0, 16)]
            if msk is not None:
                idr = jnp.where(msk, idr, 0)
                idc = jnp.where(msk, idc, 0)
            idr4 = idr * 4
            idc4 = idc * 4
            rowv = (g0 + iota) * XL
            d2 = jnp.zeros((16,), jnp.float32)
            for comp in range(3):
                xr = plsc.load_gather(xv, [idr4 + comp], mask=msk)
                xc = plsc.load_gather(xv, [idc4 + comp], mask=msk)
                rel = xr - xc
                d2 = d2 + rel * rel
                plsc.store_scatter(xrelbuf, [rowv + comp], rel, mask=msk)
            plsc.store_scatter(xrelbuf, [rowv + 3], d2, mask=msk)
        pltpu.sync_copy(xrelbuf, xrel_hbm.at[pl.ds(off * XL, GEB * XL)])
        ca.wait()
        pltpu.sync_copy(abuf, arow_hbm.at[pl.ds(off, GEB)])
        cb.wait()
        pltpu.sync_copy(bbuf, bcol_hbm.at[pl.ds(off, GEB)])
        return carry

    lax.fori_loop(0, E_PER_W // GEB, body, 0)


def _edge_body(arow_ref, bcol_ref, xrel_ref, ea_ref,
               wea_ref, be1_ref, we2_ref, be2_ref, wa_ref, ba_ref,
               wc1_ref, bc1_ref, wc2_ref, wd_ref,
               payl_ref, payr_ref, cpay_ref):
    xrel = xrel_ref[...]
    lane = lax.broadcasted_iota(jnp.int32, (1, XL), 1)
    rel = jnp.where(lane < 3, xrel, 0.0)
    d2 = xrel[:, 3:4]
    dist = jnp.sqrt(d2)
    u = _unpack_bf16_pair(arow_ref[...]) + _unpack_bf16_pair(bcol_ref[...])
    u += dist * wd_ref[...]
    u += jnp.dot(ea_ref[...], wea_ref[...], preferred_element_type=jnp.float32)
    u += be1_ref[...]
    t = u * jax.nn.sigmoid(u)
    v = jnp.dot(t.astype(jnp.bfloat16), we2_ref[...],
                preferred_element_type=jnp.float32) + be2_ref[...]
    m = v * jax.nn.sigmoid(v)
    att = jax.nn.sigmoid(
        jnp.sum(m * wa_ref[...], axis=1, keepdims=True) + ba_ref[...])
    pay = att * m
    payl_ref[...] = pay[:, :HD]
    payr_ref[...] = pay[:, HD:]
    cv = jnp.dot(m.astype(jnp.bfloat16), wc1_ref[...],
                 preferred_element_type=jnp.float32) + bc1_ref[...]
    c1 = cv * jax.nn.sigmoid(cv)
    cw = jnp.sum(c1 * wc2_ref[...], axis=1, keepdims=True)
    cpay_ref[...] = cw * rel / (dist + 1e-8)


def _edge(arow, bcol, xrel, ea, wea, be1, we2, be2, wa, ba, wc1, bc1, wc2, wd):
    be = 2000
    full = lambda shape: pl.BlockSpec(shape, lambda i: (0, 0))
    return pl.pallas_call(
        _edge_body,
        grid=(E // be,),
        in_specs=[
            pl.BlockSpec((be, HD), lambda i: (i, 0)),
            pl.BlockSpec((be, HD), lambda i: (i, 0)),
            pl.BlockSpec((be, XL), lambda i: (i, 0)),
            pl.BlockSpec((be, ED), lambda i: (i, 0)),
            full((ED, D)), full((1, D)), full((D, D)), full((1, D)),
            full((1, D)), full((1, 1)), full((D, D)), full((1, D)),
            full((1, D)), full((1, D)),
        ],
        out_specs=[
            pl.BlockSpec((be, HD), lambda i: (i, 0)),
            pl.BlockSpec((be, HD), lambda i: (i, 0)),
            pl.BlockSpec((be, XL), lambda i: (i, 0)),
        ],
        out_shape=[
            jax.ShapeDtypeStruct((E, HD), jnp.float32),
            jax.ShapeDtypeStruct((E, HD), jnp.float32),
            jax.ShapeDtypeStruct((E, XL), jnp.float32),
        ],
    )(arow, bcol, xrel, ea, wea, be1, we2, be2, wa, ba, wc1, bc1, wc2, wd)


@functools.partial(
    pl.kernel,
    mesh=_mesh,
    out_type=(
        jax.ShapeDtypeStruct((N, HD), jnp.float32),
        jax.ShapeDtypeStruct((N, HD), jnp.float32),
        jax.ShapeDtypeStruct((NW * N * 3,), jnp.float32),
    ),
    scratch_types=[
        pltpu.VMEM_SHARED((N, HD), jnp.float32),
        pltpu.VMEM((SEB,), jnp.int32),
        pltpu.VMEM((SEB, HD), jnp.float32),
        pltpu.VMEM((SEB * XL,), jnp.float32),
        pltpu.VMEM((N * 3,), jnp.float32),
        pltpu.VMEM((TMPW,), jnp.int32),
    ],
    compiler_params=_sc_params,
)
def _scatter_k(payl_hbm, payr_hbm, cpay_hbm, row_hbm,
               ml_hbm, mr_hbm, cout_hbm,
               acc_sh, ibuf, pbuf, cbuf, cvacc, tmp):
    c = lax.axis_index("c")
    s = lax.axis_index("s")
    wid = s * NC + c
    nrow0 = s * NPT

    def zpbody(k, carry):
        pbuf[k // 8, pl.ds((k % 8) * 16, 16)] = jnp.zeros((16,), jnp.float32)
        return carry
    lax.fori_loop(0, SEB * 8, zpbody, 0)

    nfull = NPT // SEB
    nrem = NPT - nfull * SEB
    for z in range(nfull):
        pltpu.sync_copy(pbuf, acc_sh.at[pl.ds(nrow0 + z * SEB, SEB)])
    pltpu.sync_copy(pbuf.at[pl.ds(0, nrem)],
                    acc_sh.at[pl.ds(nrow0 + nfull * SEB, nrem)])

    @pl.when(s == 0)
    def _():
        pltpu.sync_copy(pbuf.at[pl.ds(0, NTAIL)],
                        acc_sh.at[pl.ds(NS * NPT, NTAIL)])

    def zbody(i, carry):
        cvacc[pl.ds(i * 16, 16)] = jnp.zeros((16,), jnp.float32)
        return carry
    lax.fori_loop(0, N * 3 // 16, zbody, 0)

    plsc.subcore_barrier()

    ebase = s * E_PER_TILE
    nblk = E_PER_TILE // SEB

    def body(j, carry):
        off = ebase + j * SEB
        pltpu.sync_copy(row_hbm.at[pl.ds(off, SEB)], ibuf)

        @pl.when(c == 0)
        def _():
            pltpu.sync_copy(payl_hbm.at[pl.ds(off, SEB)], pbuf)

        @pl.when(c == 1)
        def _():
            pltpu.sync_copy(payr_hbm.at[pl.ds(off, SEB)], pbuf)

        pltpu.sync_copy(pbuf, acc_sh.at[ibuf], add=True)

        @pl.when(lax.rem(j, 2) == c)
        def _():
            pltpu.sync_copy(cpay_hbm.at[pl.ds(off * XL, SEB * XL)], cbuf)
            iota = lax.iota(jnp.int32, 16)
            for g in range(SEB // 16):
                g0 = g * 16
                idxv = ibuf[pl.ds(g0, 16)]
                ev = (g0 + iota) * XL
                cpx = plsc.load_gather(cbuf, [ev])
                cpy = plsc.load_gather(cbuf, [ev + 1])
                cpz = plsc.load_gather(cbuf, [ev + 2])
                addr = idxv * 3
                slot = jnp.bitwise_and(idxv, TMPW - 1)

                def wbody(rem):
                    plsc.store_scatter(tmp, [slot], iota, mask=rem)
                    win = jnp.logical_and(
                        plsc.load_gather(tmp, [slot], mask=rem) == iota, rem)
                    plsc.addupdate_scatter(cvacc, [addr], cpx, mask=win)
                    plsc.addupdate_scatter(cvacc, [addr + 1], cpy, mask=win)
                    plsc.addupdate_scatter(cvacc, [addr + 2], cpz, mask=win)
                    return jnp.logical_and(rem, jnp.logical_not(win))

                lax.while_loop(jnp.any, wbody, iota >= 0)

        return carry

    lax.fori_loop(0, nblk, body, 0)

    pltpu.sync_copy(cvacc, cout_hbm.at[pl.ds(wid * (N * 3), N * 3)])

    plsc.subcore_barrier()

    def _writeback(dst_hbm):
        for z in range(nfull):
            pltpu.sync_copy(acc_sh.at[pl.ds(nrow0 + z * SEB, SEB)], pbuf)
            pltpu.sync_copy(pbuf, dst_hbm.at[pl.ds(nrow0 + z * SEB, SEB)])
        pltpu.sync_copy(acc_sh.at[pl.ds(nrow0 + nfull * SEB, nrem)],
                        pbuf.at[pl.ds(0, nrem)])
        pltpu.sync_copy(pbuf.at[pl.ds(0, nrem)],
                        dst_hbm.at[pl.ds(nrow0 + nfull * SEB, nrem)])

        @pl.when(s == 0)
        def _():
            pltpu.sync_copy(acc_sh.at[pl.ds(NS * NPT, NTAIL)],
                            pbuf.at[pl.ds(0, NTAIL)])
            pltpu.sync_copy(pbuf.at[pl.ds(0, NTAIL)],
                            dst_hbm.at[pl.ds(NS * NPT, NTAIL)])

    @pl.when(c == 0)
    def _():
        _writeback(ml_hbm)

    @pl.when(c == 1)
    def _():
        _writeback(mr_hbm)


def _node_body(h_ref, ml_ref, mr_ref,
               wn1a_ref, wn1bl_ref, wn1br_ref, bn1_ref, wn2_ref, bn2_ref,
               hnew_ref):
    g = (jnp.dot(h_ref[...], wn1a_ref[...], preferred_element_type=jnp.float32)
         + jnp.dot(ml_ref[...], wn1bl_ref[...], preferred_element_type=jnp.float32)
         + jnp.dot(mr_ref[...], wn1br_ref[...], preferred_element_type=jnp.float32)
         + bn1_ref[...])
    g = g * jax.nn.sigmoid(g)
    hnew_ref[...] = (h_ref[...]
                     + jnp.dot(g, wn2_ref[...], preferred_element_type=jnp.float32)
                     + bn2_ref[...])


def _node(h, ml, mr, wn1a, wn1bl, wn1br, bn1, wn2, bn2):
    bn = 2000
    full = lambda shape: pl.BlockSpec(shape, lambda i: (0, 0))
    return pl.pallas_call(
        _node_body,
        grid=(N // bn,),
        in_specs=[
            pl.BlockSpec((bn, D), lambda i: (i, 0)),
            pl.BlockSpec((bn, HD), lambda i: (i, 0)),
            pl.BlockSpec((bn, HD), lambda i: (i, 0)),
            full((D, D)), full((HD, D)), full((HD, D)), full((1, D)),
            full((D, D)), full((1, D)),
        ],
        out_specs=pl.BlockSpec((bn, D), lambda i: (i, 0)),
        out_shape=jax.ShapeDtypeStruct((N, D), jnp.float32),
    )(h, ml, mr, wn1a, wn1bl, wn1br, bn1, wn2, bn2)


def _creduce_body(xf_ref, cout_ref, xnewf_ref):
    xnewf_ref[...] = xf_ref[...] + jnp.sum(cout_ref[...], axis=0)


def _creduce(xf, cout):
    return pl.pallas_call(
        _creduce_body,
        out_shape=jax.ShapeDtypeStruct((N * 3,), jnp.float32),
    )(xf, cout)


def kernel(h, x, edge_index, edge_attr, W_e1, b_e1, W_e2, b_e2,
           W_n1, b_n1, W_n2, b_n2, W_c1, b_c1, W_c2, W_a, b_a):
    row = edge_index[0]
    col = edge_index[1]

    w1a = W_e1[:D]
    w1b = W_e1[D:2 * D]
    wd = W_e1[2 * D:2 * D + 1]
    wea = W_e1[2 * D + 1:]
    be1 = b_e1.reshape(1, D)
    be2 = b_e2.reshape(1, D)
    wa = W_a.reshape(1, D)
    ba = b_a.reshape(1, 1)
    bc1 = b_c1.reshape(1, D)
    wc2 = W_c2.reshape(1, D)
    wn1a = W_n1[:D]
    wn1bl = W_n1[D:D + HD]
    wn1br = W_n1[D + HD:]
    bn1 = b_n1.reshape(1, D)
    bn2 = b_n2.reshape(1, D)

    x4 = jnp.pad(x, ((0, 0), (0, 1))).reshape(N * 4)

    a, b = _pre(h, w1a, w1b)
    arow, bcol, xrel = _gather_k(a, b, x4, row, col)
    payl, payr, cpay = _edge(arow, bcol, xrel.reshape(E, XL), edge_attr,
                             wea, be1, W_e2.astype(jnp.bfloat16), be2, wa, ba,
                             W_c1.astype(jnp.bfloat16), bc1, wc2, wd)
    ml, mr, cout = _scatter_k(payl, payr, cpay.reshape(E * XL), row)
    hnew = _node(h, ml, mr, wn1a, wn1bl, wn1br, bn1, W_n2, bn2)
    xnewf = _creduce(x.reshape(N * 3), cout.reshape(NW, N * 3))
    return hnew, xnewf.reshape(N, 3)


# speedup vs baseline: 3.4726x; 1.1892x over previous
import functools

import jax
import jax.numpy as jnp
from jax import lax
from jax.experimental import pallas as pl
from jax.experimental.pallas import tpu as pltpu
from jax.experimental.pallas import tpu_sc as plsc

N = 10000
E = 160000
D = 256
ED = 16
XL = 16
HD = D // 2

NC, NS = 2, 16
NW = NC * NS

GEB = 200
SEB = 80
TMPW = 512
E_PER_W = E // NW
E_PER_TILE = E // NS
NPT = 624
NTAIL = N - NS * NPT

_mesh = plsc.VectorSubcoreMesh(core_axis_name="c", subcore_axis_name="s")
_sc_params = pltpu.CompilerParams(needs_layout_passes=False)


def _pack_bf16_pair(m):
    mu = lax.bitcast_convert_type(
        m.astype(jnp.bfloat16), jnp.uint16).astype(jnp.uint32)
    packed = mu[:, :HD] | (mu[:, HD:] << 16)
    return lax.bitcast_convert_type(packed, jnp.int32)


def _unpack_bf16_pair(p):
    lo = lax.bitcast_convert_type(lax.shift_left(p, 16), jnp.float32)
    hi = lax.bitcast_convert_type(
        jnp.bitwise_and(p, jnp.int32(-65536)), jnp.float32)
    return jnp.concatenate([lo, hi], axis=1)


def _pre_body(h_ref, w1a_ref, w1b_ref, a_ref, b_ref):
    h = h_ref[...]
    a_ref[...] = _pack_bf16_pair(
        jnp.dot(h, w1a_ref[...], preferred_element_type=jnp.float32))
    b_ref[...] = _pack_bf16_pair(
        jnp.dot(h, w1b_ref[...], preferred_element_type=jnp.float32))


def _pre(h, w1a, w1b):
    bn = 2000
    return pl.pallas_call(
        _pre_body,
        grid=(N // bn,),
        in_specs=[
            pl.BlockSpec((bn, D), lambda i: (i, 0)),
            pl.BlockSpec((D, D), lambda i: (0, 0)),
            pl.BlockSpec((D, D), lambda i: (0, 0)),
        ],
        out_specs=[
            pl.BlockSpec((bn, HD), lambda i: (i, 0)),
            pl.BlockSpec((bn, HD), lambda i: (i, 0)),
        ],
        out_shape=[
            jax.ShapeDtypeStruct((N, HD), jnp.int32),
            jax.ShapeDtypeStruct((N, HD), jnp.int32),
        ],
    )(h, w1a, w1b)


@functools.partial(
    pl.kernel,
    mesh=_mesh,
    out_type=(
        jax.ShapeDtypeStruct((E, HD), jnp.int32),
        jax.ShapeDtypeStruct((E, HD), jnp.int32),
        jax.ShapeDtypeStruct((E * XL,), jnp.float32),
    ),
    scratch_types=[
        pltpu.VMEM((GEB,), jnp.int32),
        pltpu.VMEM((GEB,), jnp.int32),
        pltpu.VMEM((GEB, HD), jnp.int32),
        pltpu.VMEM((GEB, HD), jnp.int32),
        pltpu.VMEM((GEB * XL,), jnp.float32),
        pltpu.VMEM((N * 4,), jnp.float32),
        pltpu.SemaphoreType.DMA,
    ],
    compiler_params=_sc_params,
)
def _gather_k(a_hbm, b_hbm, x4_hbm, row_hbm, col_hbm,
              arow_hbm, bcol_hbm, xrel_hbm,
              rbuf, cbuf, abuf, bbuf, xrelbuf, xv, sem):
    wid = lax.axis_index("s") * NC + lax.axis_index("c")
    base = wid * E_PER_W

    pltpu.sync_copy(x4_hbm, xv)

    def zbody(i, carry):
        xrelbuf[pl.ds(i * 16, 16)] = jnp.zeros((16,), jnp.float32)
        return carry
    lax.fori_loop(0, GEB * XL // 16, zbody, 0)

    iota = lax.iota(jnp.int32, 16)
    tail_mask = iota >= 8

    def body(j, carry):
        off = base + j * GEB
        pltpu.sync_copy(row_hbm.at[pl.ds(off, GEB)], rbuf)
        pltpu.sync_copy(col_hbm.at[pl.ds(off, GEB)], cbuf)
        ca = pltpu.async_copy(a_hbm.at[rbuf], abuf, sem)
        cb = pltpu.async_copy(b_hbm.at[cbuf], bbuf, sem)
        for g in range(13):
            g0 = g * 16 if g < 12 else GEB - 16
            msk = None if g < 12 else tail_mask
            idr = rbuf[pl.ds(g0, 16)]
            idc = cbuf[pl.ds(g0, 16)]
            if msk is not None:
                idr = jnp.where(msk, idr, 0)
                idc = jnp.where(msk, idc, 0)
            idr4 = idr * 4
            idc4 = idc * 4
            rowv = (g0 + iota) * XL
            d2 = jnp.zeros((16,), jnp.float32)
            for comp in range(3):
                xr = plsc.load_gather(xv, [idr4 + comp], mask=msk)
                xc = plsc.load_gather(xv, [idc4 + comp], mask=msk)
                rel = xr - xc
                d2 = d2 + rel * rel
                plsc.store_scatter(xrelbuf, [rowv + comp], rel, mask=msk)
            plsc.store_scatter(xrelbuf, [rowv + 3], d2, mask=msk)
        pltpu.sync_copy(xrelbuf, xrel_hbm.at[pl.ds(off * XL, GEB * XL)])
        ca.wait()
        pltpu.sync_copy(abuf, arow_hbm.at[pl.ds(off, GEB)])
        cb.wait()
        pltpu.sync_copy(bbuf, bcol_hbm.at[pl.ds(off, GEB)])
        return carry

    lax.fori_loop(0, E_PER_W // GEB, body, 0)


def _edge_body(arow_ref, bcol_ref, xrel_ref, ea_ref,
               wea_ref, be1_ref, we2_ref, be2_ref, wa_ref, ba_ref,
               wc1_ref, bc1_ref, wc2_ref, wd_ref,
               payl_ref, payr_ref, cpay_ref):
    xrel = xrel_ref[...]
    lane = lax.broadcasted_iota(jnp.int32, (1, XL), 1)
    rel = jnp.where(lane < 3, xrel, 0.0)
    d2 = xrel[:, 3:4]
    dist = jnp.sqrt(d2)
    u = _unpack_bf16_pair(arow_ref[...]) + _unpack_bf16_pair(bcol_ref[...])
    u += dist * wd_ref[...]
    u += jnp.dot(ea_ref[...], wea_ref[...], preferred_element_type=jnp.float32)
    u += be1_ref[...]
    ub = u.astype(jnp.bfloat16)
    t = ub * jax.nn.sigmoid(ub)
    v = jnp.dot(t, we2_ref[...],
                preferred_element_type=jnp.float32) + be2_ref[...]
    vb = v.astype(jnp.bfloat16)
    m = vb * jax.nn.sigmoid(vb)
    mf = m.astype(jnp.float32)
    att = jax.nn.sigmoid(
        jnp.sum(mf * wa_ref[...], axis=1, keepdims=True) + ba_ref[...])
    pay = att * mf
    payl_ref[...] = pay[:, :HD]
    payr_ref[...] = pay[:, HD:]
    cv = jnp.dot(m, wc1_ref[...],
                 preferred_element_type=jnp.float32) + bc1_ref[...]
    cvb = cv.astype(jnp.bfloat16)
    c1 = (cvb * jax.nn.sigmoid(cvb)).astype(jnp.float32)
    cw = jnp.sum(c1 * wc2_ref[...], axis=1, keepdims=True)
    cpay_ref[...] = cw * rel / (dist + 1e-8)


def _edge(arow, bcol, xrel, ea, wea, be1, we2, be2, wa, ba, wc1, bc1, wc2, wd):
    be = 2000
    full = lambda shape: pl.BlockSpec(shape, lambda i: (0, 0))
    return pl.pallas_call(
        _edge_body,
        grid=(E // be,),
        in_specs=[
            pl.BlockSpec((be, HD), lambda i: (i, 0)),
            pl.BlockSpec((be, HD), lambda i: (i, 0)),
            pl.BlockSpec((be, XL), lambda i: (i, 0)),
            pl.BlockSpec((be, ED), lambda i: (i, 0)),
            full((ED, D)), full((1, D)), full((D, D)), full((1, D)),
            full((1, D)), full((1, 1)), full((D, D)), full((1, D)),
            full((1, D)), full((1, D)),
        ],
        out_specs=[
            pl.BlockSpec((be, HD), lambda i: (i, 0)),
            pl.BlockSpec((be, HD), lambda i: (i, 0)),
            pl.BlockSpec((be, XL), lambda i: (i, 0)),
        ],
        out_shape=[
            jax.ShapeDtypeStruct((E, HD), jnp.float32),
            jax.ShapeDtypeStruct((E, HD), jnp.float32),
            jax.ShapeDtypeStruct((E, XL), jnp.float32),
        ],
    )(arow, bcol, xrel, ea, wea, be1, we2, be2, wa, ba, wc1, bc1, wc2, wd)


@functools.partial(
    pl.kernel,
    mesh=_mesh,
    out_type=(
        jax.ShapeDtypeStruct((N, HD), jnp.float32),
        jax.ShapeDtypeStruct((N, HD), jnp.float32),
    ),
    scratch_types=[
        pltpu.VMEM_SHARED((N, HD), jnp.float32),
        pltpu.VMEM((SEB,), jnp.int32),
        pltpu.VMEM((SEB,), jnp.int32),
        pltpu.VMEM((SEB, HD), jnp.float32),
        pltpu.VMEM((SEB, HD), jnp.float32),
        pltpu.SemaphoreType.DMA,
        pltpu.SemaphoreType.DMA,
    ],
    compiler_params=_sc_params,
)
def _scatter_k(payl_hbm, payr_hbm, row_hbm,
               ml_hbm, mr_hbm,
               acc_sh, ibufa, ibufb, pbufa, pbufb, sema, semb):
    c = lax.axis_index("c")
    s = lax.axis_index("s")
    nrow0 = s * NPT
    pbuf = pbufa

    def zpbody(k, carry):
        pbuf[k // 8, pl.ds((k % 8) * 16, 16)] = jnp.zeros((16,), jnp.float32)
        return carry
    lax.fori_loop(0, SEB * 8, zpbody, 0)

    nfull = NPT // SEB
    nrem = NPT - nfull * SEB
    for z in range(nfull):
        pltpu.sync_copy(pbuf, acc_sh.at[pl.ds(nrow0 + z * SEB, SEB)])
    pltpu.sync_copy(pbuf.at[pl.ds(0, nrem)],
                    acc_sh.at[pl.ds(nrow0 + nfull * SEB, nrem)])

    @pl.when(s == 0)
    def _():
        pltpu.sync_copy(pbuf.at[pl.ds(0, NTAIL)],
                        acc_sh.at[pl.ds(NS * NPT, NTAIL)])

    plsc.subcore_barrier()

    ebase = s * E_PER_TILE
    nblk = E_PER_TILE // SEB

    def _pay_hbm(k_fn):
        @pl.when(c == 0)
        def _():
            k_fn(payl_hbm)

        @pl.when(c == 1)
        def _():
            k_fn(payr_hbm)

    def start_fetch(off, ib, pb, sem):
        pltpu.async_copy(row_hbm.at[pl.ds(off, SEB)], ib, sem)
        _pay_hbm(lambda p: pltpu.async_copy(p.at[pl.ds(off, SEB)], pb, sem))

    def wait_fetch(off, ib, pb, sem):
        pltpu.make_async_copy(row_hbm.at[pl.ds(off, SEB)], ib, sem).wait()
        pltpu.make_async_copy(payl_hbm.at[pl.ds(off, SEB)], pb, sem).wait()

    start_fetch(ebase, ibufa, pbufa, sema)
    start_fetch(ebase + SEB, ibufb, pbufb, semb)

    def body(j2, carry):
        for k, (ib, pb, sem) in ((0, (ibufa, pbufa, sema)),
                                 (1, (ibufb, pbufb, semb))):
            j = 2 * j2 + k
            off = ebase + j * SEB
            wait_fetch(off, ib, pb, sem)
            pltpu.sync_copy(pb, acc_sh.at[ib], add=True)

            @pl.when(j2 < nblk // 2 - 1)
            def _():
                start_fetch(off + 2 * SEB, ib, pb, sem)

        return carry

    lax.fori_loop(0, nblk // 2, body, 0)

    plsc.subcore_barrier()

    def _writeback(dst_hbm):
        for z in range(nfull):
            pltpu.sync_copy(acc_sh.at[pl.ds(nrow0 + z * SEB, SEB)], pbuf)
            pltpu.sync_copy(pbuf, dst_hbm.at[pl.ds(nrow0 + z * SEB, SEB)])
        pltpu.sync_copy(acc_sh.at[pl.ds(nrow0 + nfull * SEB, nrem)],
                        pbuf.at[pl.ds(0, nrem)])
        pltpu.sync_copy(pbuf.at[pl.ds(0, nrem)],
                        dst_hbm.at[pl.ds(nrow0 + nfull * SEB, nrem)])

        @pl.when(s == 0)
        def _():
            pltpu.sync_copy(acc_sh.at[pl.ds(NS * NPT, NTAIL)],
                            pbuf.at[pl.ds(0, NTAIL)])
            pltpu.sync_copy(pbuf.at[pl.ds(0, NTAIL)],
                            dst_hbm.at[pl.ds(NS * NPT, NTAIL)])

    @pl.when(c == 0)
    def _():
        _writeback(ml_hbm)

    @pl.when(c == 1)
    def _():
        _writeback(mr_hbm)


CPE = E // NW


@functools.partial(
    pl.kernel,
    mesh=_mesh,
    out_type=jax.ShapeDtypeStruct((NW * N * 3,), jnp.float32),
    scratch_types=[
        pltpu.VMEM((CPE,), jnp.int32),
        pltpu.VMEM((CPE * XL,), jnp.float32),
        pltpu.VMEM((N * 3,), jnp.float32),
        pltpu.VMEM((TMPW,), jnp.int32),
    ],
    compiler_params=_sc_params,
)
def _coord_k(cpay_hbm, row_hbm, cout_hbm, ibuf, cbuf, cvacc, tmp):
    wid = lax.axis_index("s") * NC + lax.axis_index("c")
    base = wid * CPE

    def zbody(i, carry):
        cvacc[pl.ds(i * 16, 16)] = jnp.zeros((16,), jnp.float32)
        return carry
    lax.fori_loop(0, N * 3 // 16, zbody, 0)

    pltpu.sync_copy(row_hbm.at[pl.ds(base, CPE)], ibuf)
    pltpu.sync_copy(cpay_hbm.at[pl.ds(base * XL, CPE * XL)], cbuf)

    iota = lax.iota(jnp.int32, 16)

    def group(g0, rem0):
        idxv = ibuf[pl.ds(g0, 16)]
        ev = (g0 + iota) * XL
        cpx = plsc.load_gather(cbuf, [ev])
        cpy = plsc.load_gather(cbuf, [ev + 1])
        cpz = plsc.load_gather(cbuf, [ev + 2])
        addr = idxv * 3
        slot = jnp.bitwise_and(idxv, TMPW - 1)

        def wbody(rem):
            plsc.store_scatter(tmp, [slot], iota, mask=rem)
            win = jnp.logical_and(
                plsc.load_gather(tmp, [slot], mask=rem) == iota, rem)
            plsc.addupdate_scatter(cvacc, [addr], cpx, mask=win)
            plsc.addupdate_scatter(cvacc, [addr + 1], cpy, mask=win)
            plsc.addupdate_scatter(cvacc, [addr + 2], cpz, mask=win)
            return jnp.logical_and(rem, jnp.logical_not(win))

        lax.while_loop(jnp.any, wbody, rem0)

    def gbody(g, carry):
        group(pl.multiple_of(g * 16, 16), iota >= 0)
        return carry
    lax.fori_loop(0, CPE // 16, gbody, 0)
    group(CPE - 16, iota >= 16 - (CPE - CPE // 16 * 16))

    pltpu.sync_copy(cvacc, cout_hbm.at[pl.ds(wid * (N * 3), N * 3)])


def _node_body(h_ref, ml_ref, mr_ref,
               wn1a_ref, wn1bl_ref, wn1br_ref, bn1_ref, wn2_ref, bn2_ref,
               hnew_ref):
    g = (jnp.dot(h_ref[...], wn1a_ref[...], preferred_element_type=jnp.float32)
         + jnp.dot(ml_ref[...], wn1bl_ref[...], preferred_element_type=jnp.float32)
         + jnp.dot(mr_ref[...], wn1br_ref[...], preferred_element_type=jnp.float32)
         + bn1_ref[...])
    g = g * jax.nn.sigmoid(g)
    hnew_ref[...] = (h_ref[...]
                     + jnp.dot(g, wn2_ref[...], preferred_element_type=jnp.float32)
                     + bn2_ref[...])


def _node(h, ml, mr, wn1a, wn1bl, wn1br, bn1, wn2, bn2):
    bn = 2000
    full = lambda shape: pl.BlockSpec(shape, lambda i: (0, 0))
    return pl.pallas_call(
        _node_body,
        grid=(N // bn,),
        in_specs=[
            pl.BlockSpec((bn, D), lambda i: (i, 0)),
            pl.BlockSpec((bn, HD), lambda i: (i, 0)),
            pl.BlockSpec((bn, HD), lambda i: (i, 0)),
            full((D, D)), full((HD, D)), full((HD, D)), full((1, D)),
            full((D, D)), full((1, D)),
        ],
        out_specs=pl.BlockSpec((bn, D), lambda i: (i, 0)),
        out_shape=jax.ShapeDtypeStruct((N, D), jnp.float32),
    )(h, ml, mr, wn1a, wn1bl, wn1br, bn1, wn2, bn2)


def _creduce_body(xf_ref, cout_ref, xnewf_ref):
    xnewf_ref[...] = xf_ref[...] + jnp.sum(cout_ref[...], axis=0)


def _creduce(xf, cout):
    return pl.pallas_call(
        _creduce_body,
        out_shape=jax.ShapeDtypeStruct((N * 3,), jnp.float32),
    )(xf, cout)


def kernel(h, x, edge_index, edge_attr, W_e1, b_e1, W_e2, b_e2,
           W_n1, b_n1, W_n2, b_n2, W_c1, b_c1, W_c2, W_a, b_a):
    row = edge_index[0]
    col = edge_index[1]

    w1a = W_e1[:D]
    w1b = W_e1[D:2 * D]
    wd = W_e1[2 * D:2 * D + 1]
    wea = W_e1[2 * D + 1:]
    be1 = b_e1.reshape(1, D)
    be2 = b_e2.reshape(1, D)
    wa = W_a.reshape(1, D)
    ba = b_a.reshape(1, 1)
    bc1 = b_c1.reshape(1, D)
    wc2 = W_c2.reshape(1, D)
    wn1a = W_n1[:D]
    wn1bl = W_n1[D:D + HD]
    wn1br = W_n1[D + HD:]
    bn1 = b_n1.reshape(1, D)
    bn2 = b_n2.reshape(1, D)

    x4 = jnp.pad(x, ((0, 0), (0, 1))).reshape(N * 4)

    a, b = _pre(h, w1a, w1b)
    arow, bcol, xrel = _gather_k(a, b, x4, row, col)
    payl, payr, cpay = _edge(arow, bcol, xrel.reshape(E, XL), edge_attr,
                             wea, be1, W_e2.astype(jnp.bfloat16), be2, wa, ba,
                             W_c1.astype(jnp.bfloat16), bc1, wc2, wd)
    ml, mr = _scatter_k(payl, payr, row)
    cout = _coord_k(cpay.reshape(E * XL), row)
    hnew = _node(h, ml, mr, wn1a, wn1bl, wn1br, bn1, W_n2, bn2)
    xnewf = _creduce(x.reshape(N * 3), cout.reshape(NW, N * 3))
    return hnew, xnewf.reshape(N, 3)


# speedup vs baseline: 4.2748x; 1.2310x over previous
import functools

import jax
import jax.numpy as jnp
from jax import lax
from jax.experimental import pallas as pl
from jax.experimental.pallas import tpu as pltpu
from jax.experimental.pallas import tpu_sc as plsc

N = 10000
E = 160000
D = 256
ED = 16
XL = 16
HD = D // 2

NC, NS = 2, 16
NW = NC * NS

GEB = 200
SEB = 80
TMPW = 512
E_PER_W = E // NW
E_PER_TILE = E // NS
NPT = 624
NTAIL = N - NS * NPT

_mesh = plsc.VectorSubcoreMesh(core_axis_name="c", subcore_axis_name="s")
_sc_params = pltpu.CompilerParams(needs_layout_passes=False)


def _pack_bf16_pair(m):
    mu = lax.bitcast_convert_type(
        m.astype(jnp.bfloat16), jnp.uint16).astype(jnp.uint32)
    packed = mu[:, :HD] | (mu[:, HD:] << 16)
    return lax.bitcast_convert_type(packed, jnp.int32)


def _unpack_bf16_pair(p):
    lo = lax.bitcast_convert_type(lax.shift_left(p, 16), jnp.float32)
    hi = lax.bitcast_convert_type(
        jnp.bitwise_and(p, jnp.int32(-65536)), jnp.float32)
    return jnp.concatenate([lo, hi], axis=1)


def _pre_body(h_ref, w1a_ref, w1b_ref, a_ref, b_ref):
    h = h_ref[...]
    a_ref[...] = _pack_bf16_pair(
        jnp.dot(h, w1a_ref[...], preferred_element_type=jnp.float32))
    b_ref[...] = _pack_bf16_pair(
        jnp.dot(h, w1b_ref[...], preferred_element_type=jnp.float32))


def _pre(h, w1a, w1b):
    bn = 2000
    return pl.pallas_call(
        _pre_body,
        grid=(N // bn,),
        in_specs=[
            pl.BlockSpec((bn, D), lambda i: (i, 0)),
            pl.BlockSpec((D, D), lambda i: (0, 0)),
            pl.BlockSpec((D, D), lambda i: (0, 0)),
        ],
        out_specs=[
            pl.BlockSpec((bn, HD), lambda i: (i, 0)),
            pl.BlockSpec((bn, HD), lambda i: (i, 0)),
        ],
        out_shape=[
            jax.ShapeDtypeStruct((N, HD), jnp.int32),
            jax.ShapeDtypeStruct((N, HD), jnp.int32),
        ],
    )(h, w1a, w1b)


@functools.partial(
    pl.kernel,
    mesh=_mesh,
    out_type=(
        jax.ShapeDtypeStruct((E, HD), jnp.int32),
        jax.ShapeDtypeStruct((E, HD), jnp.int32),
        jax.ShapeDtypeStruct((E * XL,), jnp.float32),
    ),
    scratch_types=[
        pltpu.VMEM((GEB,), jnp.int32),
        pltpu.VMEM((GEB,), jnp.int32),
        pltpu.VMEM((GEB, HD), jnp.int32),
        pltpu.VMEM((GEB, HD), jnp.int32),
        pltpu.VMEM((GEB * XL,), jnp.float32),
        pltpu.VMEM((N * 4,), jnp.float32),
        pltpu.SemaphoreType.DMA,
    ],
    compiler_params=_sc_params,
)
def _gather_k(a_hbm, b_hbm, x4_hbm, row_hbm, col_hbm,
              arow_hbm, bcol_hbm, xrel_hbm,
              rbuf, cbuf, abuf, bbuf, xrelbuf, xv, sem):
    wid = lax.axis_index("s") * NC + lax.axis_index("c")
    base = wid * E_PER_W

    pltpu.sync_copy(x4_hbm, xv)

    def zbody(i, carry):
        xrelbuf[pl.ds(i * 16, 16)] = jnp.zeros((16,), jnp.float32)
        return carry
    lax.fori_loop(0, GEB * XL // 16, zbody, 0)

    iota = lax.iota(jnp.int32, 16)
    tail_mask = iota >= 8

    def body(j, carry):
        off = base + j * GEB
        pltpu.sync_copy(row_hbm.at[pl.ds(off, GEB)], rbuf)
        pltpu.sync_copy(col_hbm.at[pl.ds(off, GEB)], cbuf)
        ca = pltpu.async_copy(a_hbm.at[rbuf], abuf, sem)
        cb = pltpu.async_copy(b_hbm.at[cbuf], bbuf, sem)
        for g in range(13):
            g0 = g * 16 if g < 12 else GEB - 16
            msk = None if g < 12 else tail_mask
            idr = rbuf[pl.ds(g0, 16)]
            idc = cbuf[pl.ds(g0, 16)]
            if msk is not None:
                idr = jnp.where(msk, idr, 0)
                idc = jnp.where(msk, idc, 0)
            idr4 = idr * 4
            idc4 = idc * 4
            rowv = (g0 + iota) * XL
            d2 = jnp.zeros((16,), jnp.float32)
            for comp in range(3):
                xr = plsc.load_gather(xv, [idr4 + comp], mask=msk)
                xc = plsc.load_gather(xv, [idc4 + comp], mask=msk)
                rel = xr - xc
                d2 = d2 + rel * rel
                plsc.store_scatter(xrelbuf, [rowv + comp], rel, mask=msk)
            plsc.store_scatter(xrelbuf, [rowv + 3], d2, mask=msk)
        pltpu.sync_copy(xrelbuf, xrel_hbm.at[pl.ds(off * XL, GEB * XL)])
        ca.wait()
        pltpu.sync_copy(abuf, arow_hbm.at[pl.ds(off, GEB)])
        cb.wait()
        pltpu.sync_copy(bbuf, bcol_hbm.at[pl.ds(off, GEB)])
        return carry

    lax.fori_loop(0, E_PER_W // GEB, body, 0)


def _edge_body(arow_ref, bcol_ref, xrel_ref, ea_ref,
               wea_ref, be1_ref, we2_ref, be2_ref, wa_ref, ba_ref,
               wc1_ref, bc1_ref, wc2_ref, wd_ref,
               payl_ref, payr_ref, cpay_ref):
    xrel = xrel_ref[...]
    lane = lax.broadcasted_iota(jnp.int32, (1, XL), 1)
    rel = jnp.where(lane < 3, xrel, 0.0)
    d2 = xrel[:, 3:4]
    dist = jnp.sqrt(d2)
    u = _unpack_bf16_pair(arow_ref[...]) + _unpack_bf16_pair(bcol_ref[...])
    u += dist * wd_ref[...]
    u += jnp.dot(ea_ref[...], wea_ref[...], preferred_element_type=jnp.float32)
    u += be1_ref[...]
    ub = u.astype(jnp.bfloat16)
    t = ub * jax.nn.sigmoid(ub)
    v = jnp.dot(t, we2_ref[...],
                preferred_element_type=jnp.float32) + be2_ref[...]
    vb = v.astype(jnp.bfloat16)
    m = vb * jax.nn.sigmoid(vb)
    mf = m.astype(jnp.float32)
    att = jax.nn.sigmoid(
        jnp.dot(m, wa_ref[...], preferred_element_type=jnp.float32)[:, :1]
        + ba_ref[...])
    pay = att * mf
    payl_ref[...] = pay[:, :HD]
    payr_ref[...] = pay[:, HD:]
    cv = jnp.dot(m, wc1_ref[...],
                 preferred_element_type=jnp.float32) + bc1_ref[...]
    cvb = cv.astype(jnp.bfloat16)
    c1 = cvb * jax.nn.sigmoid(cvb)
    cw = jnp.dot(c1, wc2_ref[...], preferred_element_type=jnp.float32)[:, :1]
    cpay_ref[...] = cw * rel / (dist + 1e-8)


def _edge(arow, bcol, xrel, ea, wea, be1, we2, be2, wa, ba, wc1, bc1, wc2, wd):
    be = 4000
    full = lambda shape: pl.BlockSpec(shape, lambda i: (0, 0))
    return pl.pallas_call(
        _edge_body,
        grid=(E // be,),
        in_specs=[
            pl.BlockSpec((be, HD), lambda i: (i, 0)),
            pl.BlockSpec((be, HD), lambda i: (i, 0)),
            pl.BlockSpec((be, XL), lambda i: (i, 0)),
            pl.BlockSpec((be, ED), lambda i: (i, 0)),
            full((ED, D)), full((1, D)), full((D, D)), full((1, D)),
            full((D, 8)), full((1, 1)), full((D, D)), full((1, D)),
            full((D, 8)), full((1, D)),
        ],
        out_specs=[
            pl.BlockSpec((be, HD), lambda i: (i, 0)),
            pl.BlockSpec((be, HD), lambda i: (i, 0)),
            pl.BlockSpec((be, XL), lambda i: (i, 0)),
        ],
        out_shape=[
            jax.ShapeDtypeStruct((E, HD), jnp.float32),
            jax.ShapeDtypeStruct((E, HD), jnp.float32),
            jax.ShapeDtypeStruct((E, XL), jnp.float32),
        ],
    )(arow, bcol, xrel, ea, wea, be1, we2, be2, wa, ba, wc1, bc1, wc2, wd)


@functools.partial(
    pl.kernel,
    mesh=_mesh,
    out_type=(
        jax.ShapeDtypeStruct((N, HD), jnp.float32),
        jax.ShapeDtypeStruct((N, HD), jnp.float32),
    ),
    scratch_types=[
        pltpu.VMEM_SHARED((N, HD), jnp.float32),
        pltpu.VMEM((SEB,), jnp.int32),
        pltpu.VMEM((SEB,), jnp.int32),
        pltpu.VMEM((SEB, HD), jnp.float32),
        pltpu.VMEM((SEB, HD), jnp.float32),
        pltpu.SemaphoreType.DMA,
        pltpu.SemaphoreType.DMA,
    ],
    compiler_params=_sc_params,
)
def _scatter_k(payl_hbm, payr_hbm, row_hbm,
               ml_hbm, mr_hbm,
               acc_sh, ibufa, ibufb, pbufa, pbufb, sema, semb):
    c = lax.axis_index("c")
    s = lax.axis_index("s")
    nrow0 = s * NPT
    pbuf = pbufa

    def zpbody(k, carry):
        pbuf[k // 8, pl.ds((k % 8) * 16, 16)] = jnp.zeros((16,), jnp.float32)
        return carry
    lax.fori_loop(0, SEB * 8, zpbody, 0)

    nfull = NPT // SEB
    nrem = NPT - nfull * SEB
    for z in range(nfull):
        pltpu.sync_copy(pbuf, acc_sh.at[pl.ds(nrow0 + z * SEB, SEB)])
    pltpu.sync_copy(pbuf.at[pl.ds(0, nrem)],
                    acc_sh.at[pl.ds(nrow0 + nfull * SEB, nrem)])

    @pl.when(s == 0)
    def _():
        pltpu.sync_copy(pbuf.at[pl.ds(0, NTAIL)],
                        acc_sh.at[pl.ds(NS * NPT, NTAIL)])

    plsc.subcore_barrier()

    ebase = s * E_PER_TILE
    nblk = E_PER_TILE // SEB

    def _pay_hbm(k_fn):
        @pl.when(c == 0)
        def _():
            k_fn(payl_hbm)

        @pl.when(c == 1)
        def _():
            k_fn(payr_hbm)

    def start_fetch(off, ib, pb, sem):
        pltpu.async_copy(row_hbm.at[pl.ds(off, SEB)], ib, sem)
        _pay_hbm(lambda p: pltpu.async_copy(p.at[pl.ds(off, SEB)], pb, sem))

    def wait_fetch(off, ib, pb, sem):
        pltpu.make_async_copy(row_hbm.at[pl.ds(off, SEB)], ib, sem).wait()
        pltpu.make_async_copy(payl_hbm.at[pl.ds(off, SEB)], pb, sem).wait()

    start_fetch(ebase, ibufa, pbufa, sema)
    start_fetch(ebase + SEB, ibufb, pbufb, semb)

    def body(j2, carry):
        for k, (ib, pb, sem) in ((0, (ibufa, pbufa, sema)),
                                 (1, (ibufb, pbufb, semb))):
            j = 2 * j2 + k
            off = ebase + j * SEB
            wait_fetch(off, ib, pb, sem)
            pltpu.sync_copy(pb, acc_sh.at[ib], add=True)

            @pl.when(j2 < nblk // 2 - 1)
            def _():
                start_fetch(off + 2 * SEB, ib, pb, sem)

        return carry

    lax.fori_loop(0, nblk // 2, body, 0)

    plsc.subcore_barrier()

    def _writeback(dst_hbm):
        for z in range(nfull):
            pltpu.sync_copy(acc_sh.at[pl.ds(nrow0 + z * SEB, SEB)], pbuf)
            pltpu.sync_copy(pbuf, dst_hbm.at[pl.ds(nrow0 + z * SEB, SEB)])
        pltpu.sync_copy(acc_sh.at[pl.ds(nrow0 + nfull * SEB, nrem)],
                        pbuf.at[pl.ds(0, nrem)])
        pltpu.sync_copy(pbuf.at[pl.ds(0, nrem)],
                        dst_hbm.at[pl.ds(nrow0 + nfull * SEB, nrem)])

        @pl.when(s == 0)
        def _():
            pltpu.sync_copy(acc_sh.at[pl.ds(NS * NPT, NTAIL)],
                            pbuf.at[pl.ds(0, NTAIL)])
            pltpu.sync_copy(pbuf.at[pl.ds(0, NTAIL)],
                            dst_hbm.at[pl.ds(NS * NPT, NTAIL)])

    @pl.when(c == 0)
    def _():
        _writeback(ml_hbm)

    @pl.when(c == 1)
    def _():
        _writeback(mr_hbm)


CPE = E // NW


@functools.partial(
    pl.kernel,
    mesh=_mesh,
    out_type=jax.ShapeDtypeStruct((NW * N * 3,), jnp.float32),
    scratch_types=[
        pltpu.VMEM((CPE,), jnp.int32),
        pltpu.VMEM((CPE * XL,), jnp.float32),
        pltpu.VMEM((N * 3,), jnp.float32),
        pltpu.VMEM((TMPW,), jnp.int32),
    ],
    compiler_params=_sc_params,
)
def _coord_k(cpay_hbm, row_hbm, cout_hbm, ibuf, cbuf, cvacc, tmp):
    wid = lax.axis_index("s") * NC + lax.axis_index("c")
    base = wid * CPE

    def zbody(i, carry):
        cvacc[pl.ds(i * 16, 16)] = jnp.zeros((16,), jnp.float32)
        return carry
    lax.fori_loop(0, N * 3 // 16, zbody, 0)

    pltpu.sync_copy(row_hbm.at[pl.ds(base, CPE)], ibuf)
    pltpu.sync_copy(cpay_hbm.at[pl.ds(base * XL, CPE * XL)], cbuf)

    iota = lax.iota(jnp.int32, 16)

    def group(g0, rem0):
        idxv = ibuf[pl.ds(g0, 16)]
        ev = (g0 + iota) * XL
        cpx = plsc.load_gather(cbuf, [ev])
        cpy = plsc.load_gather(cbuf, [ev + 1])
        cpz = plsc.load_gather(cbuf, [ev + 2])
        addr = idxv * 3
        slot = jnp.bitwise_and(idxv, TMPW - 1)

        def wbody(rem):
            plsc.store_scatter(tmp, [slot], iota, mask=rem)
            win = jnp.logical_and(
                plsc.load_gather(tmp, [slot], mask=rem) == iota, rem)
            plsc.addupdate_scatter(cvacc, [addr], cpx, mask=win)
            plsc.addupdate_scatter(cvacc, [addr + 1], cpy, mask=win)
            plsc.addupdate_scatter(cvacc, [addr + 2], cpz, mask=win)
            return jnp.logical_and(rem, jnp.logical_not(win))

        lax.while_loop(jnp.any, wbody, rem0)

    def gbody(g, carry):
        group(pl.multiple_of(g * 16, 16), iota >= 0)
        return carry
    lax.fori_loop(0, CPE // 16, gbody, 0)
    group(CPE - 16, iota >= 16 - (CPE - CPE // 16 * 16))

    pltpu.sync_copy(cvacc, cout_hbm.at[pl.ds(wid * (N * 3), N * 3)])


def _node_body(h_ref, ml_ref, mr_ref,
               wn1a_ref, wn1bl_ref, wn1br_ref, bn1_ref, wn2_ref, bn2_ref,
               hnew_ref):
    g = (jnp.dot(h_ref[...], wn1a_ref[...], preferred_element_type=jnp.float32)
         + jnp.dot(ml_ref[...], wn1bl_ref[...], preferred_element_type=jnp.float32)
         + jnp.dot(mr_ref[...], wn1br_ref[...], preferred_element_type=jnp.float32)
         + bn1_ref[...])
    g = g * jax.nn.sigmoid(g)
    hnew_ref[...] = (h_ref[...]
                     + jnp.dot(g, wn2_ref[...], preferred_element_type=jnp.float32)
                     + bn2_ref[...])


def _node(h, ml, mr, wn1a, wn1bl, wn1br, bn1, wn2, bn2):
    bn = 2000
    full = lambda shape: pl.BlockSpec(shape, lambda i: (0, 0))
    return pl.pallas_call(
        _node_body,
        grid=(N // bn,),
        in_specs=[
            pl.BlockSpec((bn, D), lambda i: (i, 0)),
            pl.BlockSpec((bn, HD), lambda i: (i, 0)),
            pl.BlockSpec((bn, HD), lambda i: (i, 0)),
            full((D, D)), full((HD, D)), full((HD, D)), full((1, D)),
            full((D, D)), full((1, D)),
        ],
        out_specs=pl.BlockSpec((bn, D), lambda i: (i, 0)),
        out_shape=jax.ShapeDtypeStruct((N, D), jnp.float32),
    )(h, ml, mr, wn1a, wn1bl, wn1br, bn1, wn2, bn2)


def _creduce_body(xf_ref, cout_ref, xnewf_ref):
    xnewf_ref[...] = xf_ref[...] + jnp.sum(cout_ref[...], axis=0)


def _creduce(xf, cout):
    return pl.pallas_call(
        _creduce_body,
        out_shape=jax.ShapeDtypeStruct((N * 3,), jnp.float32),
    )(xf, cout)


def kernel(h, x, edge_index, edge_attr, W_e1, b_e1, W_e2, b_e2,
           W_n1, b_n1, W_n2, b_n2, W_c1, b_c1, W_c2, W_a, b_a):
    row = edge_index[0]
    col = edge_index[1]

    w1a = W_e1[:D]
    w1b = W_e1[D:2 * D]
    wd = W_e1[2 * D:2 * D + 1]
    wea = W_e1[2 * D + 1:]
    be1 = b_e1.reshape(1, D)
    be2 = b_e2.reshape(1, D)
    wa = jnp.pad(W_a, ((0, 0), (0, 7))).astype(jnp.bfloat16)
    ba = b_a.reshape(1, 1)
    bc1 = b_c1.reshape(1, D)
    wc2 = jnp.pad(W_c2, ((0, 0), (0, 7))).astype(jnp.bfloat16)
    wn1a = W_n1[:D]
    wn1bl = W_n1[D:D + HD]
    wn1br = W_n1[D + HD:]
    bn1 = b_n1.reshape(1, D)
    bn2 = b_n2.reshape(1, D)

    x4 = jnp.pad(x, ((0, 0), (0, 1))).reshape(N * 4)

    a, b = _pre(h, w1a, w1b)
    arow, bcol, xrel = _gather_k(a, b, x4, row, col)
    payl, payr, cpay = _edge(arow, bcol, xrel.reshape(E, XL), edge_attr,
                             wea, be1, W_e2.astype(jnp.bfloat16), be2, wa, ba,
                             W_c1.astype(jnp.bfloat16), bc1, wc2, wd)
    ml, mr = _scatter_k(payl, payr, row)
    cout = _coord_k(cpay.reshape(E * XL), row)
    hnew = _node(h, ml, mr, wn1a, wn1bl, wn1br, bn1, W_n2, bn2)
    xnewf = _creduce(x.reshape(N * 3), cout.reshape(NW, N * 3))
    return hnew, xnewf.reshape(N, 3)
